# Initial kernel scaffold; baseline (speedup 1.0000x reference)
import jax
import jax.numpy as jnp
from jax.experimental import pallas as pl


def kernel(x, edge_index, W_fwd, att_src_fwd, att_dst_fwd, b_fwd, W_bwd, att_src_bwd, att_dst_bwd, b_bwd, W_fuse, b_fuse, gamma, beta):
    raise NotImplementedError("write your pallas kernel here")


# speedup vs baseline: 34.5732x; 34.5732x over previous
import functools

import jax
import jax.numpy as jnp
from jax import lax
from jax.experimental import pallas as pl
from jax.experimental.pallas import tpu as pltpu
from jax.experimental.pallas import tpu_sc as plsc

N = 10000
D = 128
F = 64
E = 320000
ET = E + N
NT = 16
LAN = 128
CH = 162
ET_PAD = NT * CH * LAN
N_PAD = 10240
ROWS_PER_TILE = N_PAD // NT


def _proj_body(x_ref, wf_ref, wb_ref, asf_ref, adf_ref, asb_ref, adb_ref,
               hf_ref, hb_ref, oasf_ref, oadf_ref, oasb_ref, oadb_ref):
    xb = x_ref[...]
    hf = jnp.dot(xb, wf_ref[...], preferred_element_type=jnp.float32)
    hb = jnp.dot(xb, wb_ref[...], preferred_element_type=jnp.float32)
    hf_ref[...] = hf
    hb_ref[...] = hb
    oasf_ref[...] = jnp.sum(hf * asf_ref[...], axis=1, keepdims=True)
    oadf_ref[...] = jnp.sum(hf * adf_ref[...], axis=1, keepdims=True)
    oasb_ref[...] = jnp.sum(hb * asb_ref[...], axis=1, keepdims=True)
    oadb_ref[...] = jnp.sum(hb * adb_ref[...], axis=1, keepdims=True)


def _project(x, W_fwd, W_bwd, att_src_fwd, att_dst_fwd, att_src_bwd, att_dst_bwd):
    blk = 1000
    grid = N // blk
    full = lambda shape: pl.BlockSpec(shape, lambda i: (0, 0))
    return pl.pallas_call(
        _proj_body,
        grid=(grid,),
        in_specs=[
            pl.BlockSpec((blk, D), lambda i: (i, 0)),
            full((D, F)), full((D, F)),
            full((1, F)), full((1, F)), full((1, F)), full((1, F)),
        ],
        out_specs=[
            pl.BlockSpec((blk, F), lambda i: (i, 0)),
            pl.BlockSpec((blk, F), lambda i: (i, 0)),
            pl.BlockSpec((blk, 1), lambda i: (i, 0)),
            pl.BlockSpec((blk, 1), lambda i: (i, 0)),
            pl.BlockSpec((blk, 1), lambda i: (i, 0)),
            pl.BlockSpec((blk, 1), lambda i: (i, 0)),
        ],
        out_shape=[
            jax.ShapeDtypeStruct((N, F), jnp.float32),
            jax.ShapeDtypeStruct((N, F), jnp.float32),
            jax.ShapeDtypeStruct((N, 1), jnp.float32),
            jax.ShapeDtypeStruct((N, 1), jnp.float32),
            jax.ShapeDtypeStruct((N, 1), jnp.float32),
            jax.ShapeDtypeStruct((N, 1), jnp.float32),
        ],
    )(x, W_fwd, W_bwd,
      att_src_fwd.reshape(1, F), att_dst_fwd.reshape(1, F),
      att_src_bwd.reshape(1, F), att_dst_bwd.reshape(1, F))


def _sc_edges_body(src_hbm, dst_hbm, hf_hbm, hb_hbm,
                   asf_hbm, adf_hbm, asb_hbm, adb_hbm,
                   out_f, out_b, den_f, den_b,
                   gi, si, ta, tb, rb, eb,
                   acc, dacc,
                   sidx0, sidx1, sg0, sg1, ss0, ss1):
    cid = lax.axis_index("c")
    sid = lax.axis_index("s")
    sidx = (sidx0, sidx1)
    sg = (sg0, sg1)
    ss = (ss0, ss1)
    base_row = sid * ROWS_PER_TILE
    zeros16 = jnp.zeros((16,), jnp.float32)

    def _zrow(r, _):
        for k in range(F // 16):
            rb[0, r, pl.ds(k * 16, 16)] = zeros16
        return 0

    lax.fori_loop(0, LAN, _zrow, 0)
    for k in range(LAN // 16):
        eb[0, pl.ds(k * 16, 16)] = zeros16
    for i in range(ROWS_PER_TILE // LAN):
        pltpu.sync_copy(rb.at[0], acc.at[pl.ds(base_row + i * LAN, LAN)])
        pltpu.sync_copy(eb.at[0], dacc.at[pl.ds(base_row + i * LAN, LAN)])
    plsc.subcore_barrier()

    def run_direction(gsl, ssl, ta_hbm, tb_hbm, tab_hbm):
        pltpu.sync_copy(ta_hbm, ta)
        pltpu.sync_copy(tb_hbm, tb)
        ebase = sid * (CH * LAN)

        def start_idx(j, s):
            pltpu.async_copy(gsl.at[sid, j], gi.at[s], sidx[s])
            pltpu.async_copy(ssl.at[sid, j], si.at[s], sidx[s])

        def wait_idx(j, s):
            pltpu.make_async_copy(gsl.at[sid, j], gi.at[s], sidx[s]).wait()
            pltpu.make_async_copy(ssl.at[sid, j], si.at[s], sidx[s]).wait()

        def start_gather(s):
            pltpu.async_copy(tab_hbm.at[gi.at[s]], rb.at[s], sg[s])

        def wait_gather(s):
            pltpu.make_async_copy(tab_hbm.at[gi.at[s]], rb.at[s], sg[s]).wait()

        def start_scat(s):
            pltpu.async_copy(rb.at[s], acc.at[si.at[s]], ss[s], add=True)
            pltpu.async_copy(eb.at[s], dacc.at[si.at[s]], ss[s], add=True)

        def wait_scat(s):
            pltpu.make_async_copy(rb.at[s], acc.at[si.at[s]], ss[s]).wait()
            pltpu.make_async_copy(eb.at[s], dacc.at[si.at[s]], ss[s]).wait()

        def compute_e(j, s):
            for g in range(LAN // 16):
                gv = gi[s, pl.ds(g * 16, 16)]
                sv = si[s, pl.ds(g * 16, 16)]
                ids = ebase + j * LAN + g * 16 + lax.iota(jnp.int32, 16)
                e = plsc.load_gather(ta, [gv]) + plsc.load_gather(tb, [sv])
                e = jnp.exp(jnp.maximum(e, 0.2 * e))
                eb[s, pl.ds(g * 16, 16)] = jnp.where(ids < ET, e, 0.0)

        def scale(s):
            def sbody(rg, _):
                ev = eb[s, pl.ds(rg * 16, 16)]
                for r16 in range(16):
                    m = jnp.full((16,), ev[r16], jnp.float32)
                    r = rg * 16 + r16
                    for k in range(F // 16):
                        rb[s, r, pl.ds(k * 16, 16)] = rb[s, r, pl.ds(k * 16, 16)] * m
                return 0

            lax.fori_loop(0, LAN // 16, sbody, 0)

        start_idx(0, 0)
        start_idx(1, 1)
        wait_idx(0, 0)
        start_gather(0)
        wait_idx(1, 1)
        start_gather(1)

        def chunk_pair(jj, _):
            a = 2 * jj
            b2 = a + 1
            compute_e(a, 0)
            wait_gather(0)
            scale(0)
            start_scat(0)
            compute_e(b2, 1)
            wait_scat(0)

            @pl.when(jj < (CH // 2) - 1)
            def _():
                start_idx(a + 2, 0)

            wait_gather(1)
            scale(1)
            start_scat(1)

            @pl.when(jj < (CH // 2) - 1)
            def _():
                wait_idx(a + 2, 0)
                start_gather(0)

            wait_scat(1)

            @pl.when(jj < (CH // 2) - 1)
            def _():
                start_idx(b2 + 2, 1)
                wait_idx(b2 + 2, 1)
                start_gather(1)

            return 0

        lax.fori_loop(0, CH // 2, chunk_pair, 0)

    @pl.when(cid == 0)
    def _():
        run_direction(src_hbm, dst_hbm, asf_hbm, adf_hbm, hf_hbm)

    @pl.when(cid == 1)
    def _():
        run_direction(dst_hbm, src_hbm, asb_hbm, adb_hbm, hb_hbm)

    plsc.subcore_barrier()

    @pl.when(cid == 0)
    def _():
        pltpu.sync_copy(acc.at[pl.ds(base_row, ROWS_PER_TILE)],
                        out_f.at[pl.ds(base_row, ROWS_PER_TILE)])
        pltpu.sync_copy(dacc.at[pl.ds(base_row, ROWS_PER_TILE)],
                        den_f.at[pl.ds(base_row, ROWS_PER_TILE)])

    @pl.when(cid == 1)
    def _():
        pltpu.sync_copy(acc.at[pl.ds(base_row, ROWS_PER_TILE)],
                        out_b.at[pl.ds(base_row, ROWS_PER_TILE)])
        pltpu.sync_copy(dacc.at[pl.ds(base_row, ROWS_PER_TILE)],
                        den_b.at[pl.ds(base_row, ROWS_PER_TILE)])


def _sc_edges(src_sl, dst_sl, hf, hb, asf, adf, asb, adb):
    mesh = plsc.VectorSubcoreMesh(core_axis_name="c", subcore_axis_name="s")
    kern = functools.partial(
        pl.kernel,
        out_type=[
            jax.ShapeDtypeStruct((N_PAD, F), jnp.float32),
            jax.ShapeDtypeStruct((N_PAD, F), jnp.float32),
            jax.ShapeDtypeStruct((N_PAD,), jnp.float32),
            jax.ShapeDtypeStruct((N_PAD,), jnp.float32),
        ],
        mesh=mesh,
        compiler_params=pltpu.CompilerParams(
            needs_layout_passes=False, use_tc_tiling_on_sc=False),
        scratch_types=[
            pltpu.VMEM((2, LAN), jnp.int32),
            pltpu.VMEM((2, LAN), jnp.int32),
            pltpu.VMEM((N,), jnp.float32),
            pltpu.VMEM((N,), jnp.float32),
            pltpu.VMEM((2, LAN, F), jnp.float32),
            pltpu.VMEM((2, LAN), jnp.float32),
            pltpu.VMEM_SHARED((N_PAD, F), jnp.float32),
            pltpu.VMEM_SHARED((N_PAD,), jnp.float32),
        ] + [pltpu.SemaphoreType.DMA] * 6,
    )(_sc_edges_body)
    return kern(src_sl, dst_sl, hf, hb, asf, adf, asb, adb)


def _fuse_body(of_ref, ob_ref, df_ref, db_ref, bf_ref, bb_ref, wfu_ref,
               bfu_ref, fused_ref, psum_ref, psq_ref):
    df = df_ref[...]
    db = db_ref[...]
    fwd = of_ref[...] / (df + 1e-16) + bf_ref[...]
    bwd = ob_ref[...] / (db + 1e-16) + bb_ref[...]
    comb = jnp.concatenate([fwd, bwd], axis=1)
    fused = jnp.dot(comb, wfu_ref[...], preferred_element_type=jnp.float32)
    fused = fused + bfu_ref[...]
    fused_ref[...] = fused
    valid = df > 0.0
    fsel = jnp.where(valid, fused, 0.0)
    psum_ref[...] = jnp.sum(fsel, axis=0, keepdims=True)[None]
    psq_ref[...] = jnp.sum(fsel * fsel, axis=0, keepdims=True)[None]


def _norm_body(fused_ref, psum_ref, psq_ref, g_ref, b_ref, out_ref):
    s = jnp.sum(psum_ref[...], axis=(0, 1))
    sq = jnp.sum(psq_ref[...], axis=(0, 1))
    mu = s / float(N)
    var = sq / float(N) - mu * mu
    inv = g_ref[...] / jnp.sqrt(var + 1e-5)
    y = (fused_ref[...] - mu) * inv + b_ref[...]
    out_ref[...] = jnp.maximum(y, 0.0)


def _fuse_norm(of, ob, df, db, b_fwd, b_bwd, W_fuse, b_fuse, gamma, beta):
    blk = 1024
    grid = N_PAD // blk
    full2 = lambda shape: pl.BlockSpec(shape, lambda i: (0, 0))
    fused, psum, psq = pl.pallas_call(
        _fuse_body,
        grid=(grid,),
        in_specs=[
            pl.BlockSpec((blk, F), lambda i: (i, 0)),
            pl.BlockSpec((blk, F), lambda i: (i, 0)),
            pl.BlockSpec((blk, 1), lambda i: (i, 0)),
            pl.BlockSpec((blk, 1), lambda i: (i, 0)),
            full2((1, F)), full2((1, F)),
            full2((D, D)), full2((1, D)),
        ],
        out_specs=[
            pl.BlockSpec((blk, D), lambda i: (i, 0)),
            pl.BlockSpec((1, 1, D), lambda i: (i, 0, 0)),
            pl.BlockSpec((1, 1, D), lambda i: (i, 0, 0)),
        ],
        out_shape=[
            jax.ShapeDtypeStruct((N_PAD, D), jnp.float32),
            jax.ShapeDtypeStruct((grid, 1, D), jnp.float32),
            jax.ShapeDtypeStruct((grid, 1, D), jnp.float32),
        ],
    )(of, ob, df.reshape(N_PAD, 1), db.reshape(N_PAD, 1),
      b_fwd.reshape(1, F), b_bwd.reshape(1, F), W_fuse, b_fuse.reshape(1, D))

    out = pl.pallas_call(
        _norm_body,
        grid=(grid,),
        in_specs=[
            pl.BlockSpec((blk, D), lambda i: (i, 0)),
            pl.BlockSpec((grid, 1, D), lambda i: (0, 0, 0)),
            pl.BlockSpec((grid, 1, D), lambda i: (0, 0, 0)),
            full2((1, D)), full2((1, D)),
        ],
        out_specs=pl.BlockSpec((blk, D), lambda i: (i, 0)),
        out_shape=jax.ShapeDtypeStruct((N_PAD, D), jnp.float32),
    )(fused, psum, psq, gamma.reshape(1, D), beta.reshape(1, D))
    return out[:N]


def kernel(x, edge_index, W_fwd, att_src_fwd, att_dst_fwd, b_fwd,
           W_bwd, att_src_bwd, att_dst_bwd, b_bwd, W_fuse, b_fuse, gamma, beta):
    src = edge_index[0]
    dst = edge_index[1]
    loop = jnp.arange(N, dtype=src.dtype)
    padz = jnp.zeros((ET_PAD - ET,), src.dtype)
    src_sl = jnp.concatenate([src, loop, padz]).reshape(NT, CH, LAN)
    dst_sl = jnp.concatenate([dst, loop, padz]).reshape(NT, CH, LAN)

    hf, hb, asf, adf, asb, adb = _project(
        x, W_fwd, W_bwd, att_src_fwd, att_dst_fwd, att_src_bwd, att_dst_bwd)

    of, ob, df, db = _sc_edges(
        src_sl, dst_sl, hf, hb,
        asf.reshape(N), adf.reshape(N), asb.reshape(N), adb.reshape(N))

    return _fuse_norm(of, ob, df, db, b_fwd, b_bwd, W_fuse, b_fuse,
                      gamma, beta)
